# baseline (speedup 1.0000x reference)
import functools

import jax
import jax.numpy as jnp
from jax import lax
from jax.experimental import pallas as pl
from jax.experimental.pallas import tpu as pltpu
from jax.experimental.pallas import tpu_sc as plsc

VOCAB = 100000
EMBED = 64
CTX = 10
HIDDEN = 128
NTOK = 2 * CTX
FEAT = NTOK * EMBED

BV = 32768
NV = (VOCAB + BV - 1) // BV


def _sc_gather(emb, idx):
    mesh = plsc.VectorSubcoreMesh(core_axis_name="c", subcore_axis_name="s")

    @functools.partial(
        pl.kernel,
        mesh=mesh,
        out_type=jax.ShapeDtypeStruct((NTOK, EMBED), jnp.float32),
        scratch_types=[
            pltpu.VMEM((32,), jnp.int32),
            pltpu.VMEM((NTOK, EMBED), jnp.float32),
            pltpu.SemaphoreType.DMA,
        ],
    )
    def gather_kernel(emb_hbm, idx_hbm, out_hbm, idx_v, rows_v, sem):
        wid = lax.axis_index("s") * 2 + lax.axis_index("c")

        @pl.when(wid == 0)
        def _():
            pltpu.sync_copy(idx_hbm, idx_v.at[pl.ds(0, NTOK)])
            lo = idx_v[pl.ds(0, 16)]
            hi = idx_v[pl.ds(16, 16)]
            rows = [lo[i] for i in range(16)] + [hi[i] for i in range(NTOK - 16)]
            copies = [
                pltpu.make_async_copy(emb_hbm.at[rows[i]], rows_v.at[i], sem)
                for i in range(NTOK)
            ]
            for c in copies:
                c.start()
            for c in copies:
                c.wait()
            pltpu.sync_copy(rows_v, out_hbm)

    return gather_kernel(emb, idx)


def _mlp_body(e_ref, w1_ref, b1_ref, w2_ref, b2_ref, out_ref, h_ref):
    @pl.when(pl.program_id(0) == 0)
    def _():
        h = lax.dot_general(
            e_ref[...], w1_ref[...], (((1,), (1,)), ((), ())),
            preferred_element_type=jnp.float32)
        h_ref[...] = jnp.maximum(h + b1_ref[...][None, :], 0.0)

    out_ref[...] = lax.dot_general(
        h_ref[...], w2_ref[...], (((1,), (1,)), ((), ())),
        preferred_element_type=jnp.float32) + b2_ref[...][None, :]


def _tc_mlp(e_flat, W1, b1, W2, b2):
    return pl.pallas_call(
        _mlp_body,
        grid=(NV,),
        in_specs=[
            pl.BlockSpec((1, FEAT), lambda i: (0, 0)),
            pl.BlockSpec((HIDDEN, FEAT), lambda i: (0, 0)),
            pl.BlockSpec((HIDDEN,), lambda i: (0,)),
            pl.BlockSpec((BV, HIDDEN), lambda i: (i, 0)),
            pl.BlockSpec((BV,), lambda i: (i,)),
        ],
        out_specs=pl.BlockSpec((1, BV), lambda i: (0, i)),
        out_shape=jax.ShapeDtypeStruct((1, VOCAB), jnp.float32),
        scratch_shapes=[pltpu.VMEM((1, HIDDEN), jnp.float32)],
    )(e_flat, W1, b1, W2, b2)


def kernel(x, emb, W1, b1, W2, b2):
    e = _sc_gather(emb, x.astype(jnp.int32))
    return _tc_mlp(e.reshape(1, FEAT), W1, b1, W2, b2)


# speedup vs baseline: 1.0348x; 1.0348x over previous
import functools

import jax
import jax.numpy as jnp
from jax import lax
from jax.experimental import pallas as pl
from jax.experimental.pallas import tpu as pltpu
from jax.experimental.pallas import tpu_sc as plsc

VOCAB = 100000
EMBED = 64
CTX = 10
HIDDEN = 128
NTOK = 2 * CTX
FEAT = NTOK * EMBED

BV = 6400
NG = (VOCAB + 2 * BV - 1) // (2 * BV)


def _sc_gather(emb, idx):
    mesh = plsc.VectorSubcoreMesh(core_axis_name="c", subcore_axis_name="s")

    @functools.partial(
        pl.kernel,
        mesh=mesh,
        out_type=jax.ShapeDtypeStruct((NTOK, EMBED), jnp.float32),
        scratch_types=[
            pltpu.VMEM((32,), jnp.int32),
            pltpu.VMEM((NTOK, EMBED), jnp.float32),
            pltpu.SemaphoreType.DMA,
        ],
    )
    def gather_kernel(emb_hbm, idx_hbm, out_hbm, idx_v, rows_v, sem):
        wid = lax.axis_index("s") * 2 + lax.axis_index("c")

        @pl.when(wid == 0)
        def _():
            pltpu.sync_copy(idx_hbm, idx_v.at[pl.ds(0, NTOK)])
            lo = idx_v[pl.ds(0, 16)]
            hi = idx_v[pl.ds(16, 16)]
            rows = [lo[i] for i in range(16)] + [hi[i] for i in range(NTOK - 16)]
            copies = [
                pltpu.make_async_copy(emb_hbm.at[rows[i]], rows_v.at[i], sem)
                for i in range(NTOK)
            ]
            for c in copies:
                c.start()
            for c in copies:
                c.wait()
            pltpu.sync_copy(rows_v, out_hbm)

    return gather_kernel(emb, idx)


def _mlp_body(e_ref, w1_ref, b1_ref, w2a_ref, w2b_ref, b2_ref, out_ref, h_ref):
    @pl.when(pl.program_id(0) == 0)
    def _():
        h = lax.dot_general(
            e_ref[...], w1_ref[...], (((1,), (1,)), ((), ())),
            preferred_element_type=jnp.float32)
        h_ref[...] = jnp.maximum(h + b1_ref[...][None, :], 0.0)

    out_ref[:, :BV] = lax.dot_general(
        h_ref[...], w2a_ref[...], (((1,), (1,)), ((), ())),
        preferred_element_type=jnp.float32) + b2_ref[:, :BV]
    out_ref[:, BV:] = lax.dot_general(
        h_ref[...], w2b_ref[...], (((1,), (1,)), ((), ())),
        preferred_element_type=jnp.float32) + b2_ref[:, BV:]


def _tc_mlp(e_flat, W1, b1, W2, b2):
    return pl.pallas_call(
        _mlp_body,
        grid=(NG,),
        in_specs=[
            pl.BlockSpec((1, FEAT), lambda i: (0, 0)),
            pl.BlockSpec((HIDDEN, FEAT), lambda i: (0, 0)),
            pl.BlockSpec((HIDDEN,), lambda i: (0,)),
            pl.BlockSpec((BV, HIDDEN), lambda i: (2 * i, 0)),
            pl.BlockSpec((BV, HIDDEN), lambda i: (2 * i + 1, 0)),
            pl.BlockSpec((1, 2 * BV), lambda i: (0, i)),
        ],
        out_specs=pl.BlockSpec((1, 2 * BV), lambda i: (0, i)),
        out_shape=jax.ShapeDtypeStruct((1, VOCAB), jnp.float32),
        scratch_shapes=[pltpu.VMEM((1, HIDDEN), jnp.float32)],
    )(e_flat, W1, b1, W2, W2, b2.reshape(1, VOCAB))


def kernel(x, emb, W1, b1, W2, b2):
    e = _sc_gather(emb, x.astype(jnp.int32))
    return _tc_mlp(e.reshape(1, FEAT), W1, b1, W2, b2)
